# baseline (speedup 1.0000x reference)
import jax
import jax.numpy as jnp
from jax import lax
from jax.experimental import pallas as pl
from jax.experimental.pallas import tpu as pltpu
from jax.experimental.pallas import tpu_sc as plsc

BATCH = 4096
MAX_SEQ = 200
EMBED = 64
LANES = 16

_info = plsc.get_sparse_core_info()
NUM_CORES = _info.num_cores
NUM_SUBCORES = _info.num_subcores
NUM_WORKERS = NUM_CORES * NUM_SUBCORES

TOTAL_ROWS = BATCH * MAX_SEQ
ROWS_PER_WORKER = TOTAL_ROWS // NUM_WORKERS
CHUNK = 400
CHUNKS_PER_WORKER = ROWS_PER_WORKER // CHUNK
VECS_PER_ROW = EMBED // LANES


def _sc_body(table_hbm, idx_hbm, pos_hbm, out_hbm,
             idx_v, rows_v, pos_v, gsem):
    wid = lax.axis_index("s") * NUM_CORES + lax.axis_index("c")
    base = wid * ROWS_PER_WORKER

    pltpu.sync_copy(pos_hbm, pos_v)

    def chunk_body(g, carry):
        start = base + g * CHUNK
        pltpu.sync_copy(idx_hbm.at[pl.ds(start, CHUNK)], idx_v)
        pltpu.async_copy(table_hbm.at[idx_v], rows_v, gsem).wait()

        def add_body(r, c2):
            for j in range(VECS_PER_ROW):
                sl = pl.ds(j * LANES, LANES)
                rows_v[r, sl] = rows_v[r, sl] + pos_v[r, sl]
            return c2

        lax.fori_loop(0, CHUNK, add_body, 0, unroll=2)
        pltpu.sync_copy(rows_v, out_hbm.at[pl.ds(start, CHUNK)])
        return carry

    lax.fori_loop(0, CHUNKS_PER_WORKER, chunk_body, 0)


@jax.jit
def _embed(idx_flat, token_table, pos_tiled):
    mesh = plsc.VectorSubcoreMesh(core_axis_name="c", subcore_axis_name="s")
    run = pl.kernel(
        _sc_body,
        out_type=jax.ShapeDtypeStruct((TOTAL_ROWS, EMBED), jnp.float32),
        mesh=mesh,
        scratch_types=[
            pltpu.VMEM((CHUNK,), jnp.int32),
            pltpu.VMEM((CHUNK, EMBED), jnp.float32),
            pltpu.VMEM((CHUNK, EMBED), jnp.float32),
            pltpu.SemaphoreType.DMA,
        ],
        compiler_params=pltpu.CompilerParams(use_tc_tiling_on_sc=False),
    )
    return run(token_table, idx_flat, pos_tiled)


def kernel(inputs, token_table, position_table):
    idx_flat = inputs.reshape(-1).astype(jnp.int32)
    pos_tiled = jnp.tile(position_table, (CHUNK // MAX_SEQ, 1))
    out = _embed(idx_flat, token_table, pos_tiled)
    return out.reshape(BATCH, MAX_SEQ, EMBED)


# speedup vs baseline: 1.3918x; 1.3918x over previous
import jax
import jax.numpy as jnp
from jax import lax
from jax.experimental import pallas as pl
from jax.experimental.pallas import tpu as pltpu
from jax.experimental.pallas import tpu_sc as plsc

BATCH = 4096
MAX_SEQ = 200
EMBED = 64
LANES = 16

_info = plsc.get_sparse_core_info()
NUM_CORES = _info.num_cores
NUM_SUBCORES = _info.num_subcores
NUM_WORKERS = NUM_CORES * NUM_SUBCORES

TOTAL_ROWS = BATCH * MAX_SEQ
ROWS_PER_WORKER = TOTAL_ROWS // NUM_WORKERS
CHUNK = 400
NCHUNKS = ROWS_PER_WORKER // CHUNK
VECS_PER_ROW = EMBED // LANES


def _sc_body(table_hbm, idx_hbm, pos_hbm, out_hbm,
             idx0, idx1, rows0, rows1, pos_v,
             isem0, isem1, gsem0, gsem1, osem0, osem1):
    wid = lax.axis_index("s") * NUM_CORES + lax.axis_index("c")
    base = wid * ROWS_PER_WORKER

    bufs = ((idx0, rows0, isem0, gsem0, osem0),
            (idx1, rows1, isem1, gsem1, osem1))

    def start_idx(g, b):
        idx_v, _, isem, _, _ = bufs[b]
        pltpu.async_copy(idx_hbm.at[pl.ds(base + g * CHUNK, CHUNK)], idx_v, isem)

    def wait_idx(b):
        idx_v, _, isem, _, _ = bufs[b]
        pltpu.make_async_copy(idx_hbm.at[pl.ds(base, CHUNK)], idx_v, isem).wait()

    def start_gather(b):
        idx_v, rows_v, _, gsem, _ = bufs[b]
        pltpu.async_copy(table_hbm.at[idx_v], rows_v, gsem)

    def wait_gather(b):
        idx_v, rows_v, _, gsem, _ = bufs[b]
        pltpu.make_async_copy(table_hbm.at[idx_v], rows_v, gsem).wait()

    def start_scatter(g, b):
        _, rows_v, _, _, osem = bufs[b]
        pltpu.async_copy(rows_v, out_hbm.at[pl.ds(base + g * CHUNK, CHUNK)], osem)

    def wait_scatter(b):
        _, rows_v, _, _, osem = bufs[b]
        pltpu.make_async_copy(rows_v, out_hbm.at[pl.ds(base, CHUNK)], osem).wait()

    def add_pos(b):
        _, rows_v, _, _, _ = bufs[b]

        def add_body(r, c):
            for j in range(VECS_PER_ROW):
                sl = pl.ds(j * LANES, LANES)
                plsc.addupdate(rows_v.at[r, sl], pos_v[r, sl])
            return c

        lax.fori_loop(0, CHUNK, add_body, 0, unroll=4)

    pltpu.sync_copy(pos_hbm, pos_v)
    start_idx(0, 0)
    start_idx(1, 1)
    wait_idx(0)
    start_gather(0)

    def pair_body(i, carry):
        for b in (0, 1):
            g = 2 * i + b
            wait_gather(b)
            @pl.when(g + 2 < NCHUNKS)
            def _():
                start_idx(g + 2, b)
            @pl.when(g >= 1)
            def _():
                wait_scatter(1 - b)
            @pl.when(g + 1 < NCHUNKS)
            def _():
                wait_idx(1 - b)
                start_gather(1 - b)
            add_pos(b)
            start_scatter(g, b)
        return carry

    lax.fori_loop(0, NCHUNKS // 2, pair_body, 0)
    wait_scatter((NCHUNKS - 1) % 2)


@jax.jit
def _embed(idx_flat, token_table, pos_tiled):
    mesh = plsc.VectorSubcoreMesh(core_axis_name="c", subcore_axis_name="s")
    run = pl.kernel(
        _sc_body,
        out_type=jax.ShapeDtypeStruct((TOTAL_ROWS, EMBED), jnp.float32),
        mesh=mesh,
        scratch_types=[
            pltpu.VMEM((CHUNK,), jnp.int32),
            pltpu.VMEM((CHUNK,), jnp.int32),
            pltpu.VMEM((CHUNK, EMBED), jnp.float32),
            pltpu.VMEM((CHUNK, EMBED), jnp.float32),
            pltpu.VMEM((CHUNK, EMBED), jnp.float32),
            pltpu.SemaphoreType.DMA,
            pltpu.SemaphoreType.DMA,
            pltpu.SemaphoreType.DMA,
            pltpu.SemaphoreType.DMA,
            pltpu.SemaphoreType.DMA,
            pltpu.SemaphoreType.DMA,
        ],
        compiler_params=pltpu.CompilerParams(use_tc_tiling_on_sc=False),
    )
    return run(token_table, idx_flat, pos_tiled)


def kernel(inputs, token_table, position_table):
    idx_flat = inputs.reshape(-1).astype(jnp.int32)
    pos_tiled = jnp.tile(position_table, (CHUNK // MAX_SEQ, 1))
    out = _embed(idx_flat, token_table, pos_tiled)
    return out.reshape(BATCH, MAX_SEQ, EMBED)


# speedup vs baseline: 1.3960x; 1.0030x over previous
import jax
import jax.numpy as jnp
from jax import lax
from jax.experimental import pallas as pl
from jax.experimental.pallas import tpu as pltpu
from jax.experimental.pallas import tpu_sc as plsc

BATCH = 4096
MAX_SEQ = 200
EMBED = 64
LANES = 16

_info = plsc.get_sparse_core_info()
NUM_CORES = _info.num_cores
NUM_SUBCORES = _info.num_subcores
NUM_WORKERS = NUM_CORES * NUM_SUBCORES

TOTAL_ROWS = BATCH * MAX_SEQ
ROWS_PER_WORKER = TOTAL_ROWS // NUM_WORKERS
CHUNK = 400
NCHUNKS = ROWS_PER_WORKER // CHUNK
VECS_PER_ROW = EMBED // LANES


def _sc_body(table_hbm, idx_hbm, pos_hbm, out_hbm,
             idx0, idx1, rows0, rows1, pos_v,
             isem0, isem1, gsem0, gsem1, osem0, osem1):
    wid = lax.axis_index("s") * NUM_CORES + lax.axis_index("c")
    base = wid * ROWS_PER_WORKER

    bufs = ((idx0, rows0, isem0, gsem0, osem0),
            (idx1, rows1, isem1, gsem1, osem1))

    def start_idx(g, b):
        idx_v, _, isem, _, _ = bufs[b]
        pltpu.async_copy(idx_hbm.at[pl.ds(base + g * CHUNK, CHUNK)], idx_v, isem)

    def wait_idx(b):
        idx_v, _, isem, _, _ = bufs[b]
        pltpu.make_async_copy(idx_hbm.at[pl.ds(base, CHUNK)], idx_v, isem).wait()

    def start_gather(b):
        idx_v, rows_v, _, gsem, _ = bufs[b]
        pltpu.async_copy(table_hbm.at[idx_v], rows_v, gsem)

    def wait_gather(b):
        idx_v, rows_v, _, gsem, _ = bufs[b]
        pltpu.make_async_copy(table_hbm.at[idx_v], rows_v, gsem).wait()

    def start_scatter(g, b):
        _, rows_v, _, _, osem = bufs[b]
        pltpu.async_copy(rows_v, out_hbm.at[pl.ds(base + g * CHUNK, CHUNK)], osem)

    def wait_scatter(b):
        _, rows_v, _, _, osem = bufs[b]
        pltpu.make_async_copy(rows_v, out_hbm.at[pl.ds(base, CHUNK)], osem).wait()

    def add_pos(b):
        _, rows_v, _, _, _ = bufs[b]

        @plsc.parallel_loop(0, CHUNK, 1, unroll=8)
        def _body(r):
            for j in range(VECS_PER_ROW):
                sl = pl.ds(j * LANES, LANES)
                plsc.addupdate(rows_v.at[r, sl], pos_v[r, sl])

    pltpu.sync_copy(pos_hbm, pos_v)
    start_idx(0, 0)
    start_idx(1, 1)
    wait_idx(0)
    start_gather(0)

    def pair_body(i, carry):
        for b in (0, 1):
            g = 2 * i + b
            wait_gather(b)
            @pl.when(g + 2 < NCHUNKS)
            def _():
                start_idx(g + 2, b)
            @pl.when(g >= 1)
            def _():
                wait_scatter(1 - b)
            @pl.when(g + 1 < NCHUNKS)
            def _():
                wait_idx(1 - b)
                start_gather(1 - b)
            add_pos(b)
            start_scatter(g, b)
        return carry

    lax.fori_loop(0, NCHUNKS // 2, pair_body, 0)
    wait_scatter((NCHUNKS - 1) % 2)


@jax.jit
def _embed(idx_flat, token_table, pos_tiled):
    mesh = plsc.VectorSubcoreMesh(core_axis_name="c", subcore_axis_name="s")
    run = pl.kernel(
        _sc_body,
        out_type=jax.ShapeDtypeStruct((TOTAL_ROWS, EMBED), jnp.float32),
        mesh=mesh,
        scratch_types=[
            pltpu.VMEM((CHUNK,), jnp.int32),
            pltpu.VMEM((CHUNK,), jnp.int32),
            pltpu.VMEM((CHUNK, EMBED), jnp.float32),
            pltpu.VMEM((CHUNK, EMBED), jnp.float32),
            pltpu.VMEM((CHUNK, EMBED), jnp.float32),
            pltpu.SemaphoreType.DMA,
            pltpu.SemaphoreType.DMA,
            pltpu.SemaphoreType.DMA,
            pltpu.SemaphoreType.DMA,
            pltpu.SemaphoreType.DMA,
            pltpu.SemaphoreType.DMA,
        ],
        compiler_params=pltpu.CompilerParams(use_tc_tiling_on_sc=False),
    )
    return run(token_table, idx_flat, pos_tiled)


def kernel(inputs, token_table, position_table):
    idx_flat = inputs.reshape(-1).astype(jnp.int32)
    pos_tiled = jnp.tile(position_table, (CHUNK // MAX_SEQ, 1))
    out = _embed(idx_flat, token_table, pos_tiled)
    return out.reshape(BATCH, MAX_SEQ, EMBED)
